# baseline (speedup 1.0000x reference)
import jax
import jax.numpy as jnp
from jax import lax
from jax.experimental import pallas as pl
from jax.experimental.pallas import tpu as pltpu
from jax.experimental.pallas import tpu_sc as plsc

_FIELD0 = 1000000
_EMBED = 32
_BATCH = 16384
_NC = 2
_NS = 16
_NW = _NC * _NS
_BPW = _BATCH // _NW
_CHUNK = 128
_NCHUNK = _BPW // _CHUNK


def _gmf_body(x_hbm, table_hbm, w_hbm, b_hbm, out_hbm,
              xv, uidx_v, itidx_v, urows_v, itrows_v, out_v, wv, bv, sem):
    wid = lax.axis_index("s") * _NC + lax.axis_index("c")
    base = wid * _BPW

    pltpu.sync_copy(x_hbm.at[pl.ds(base, _BPW)], xv)
    pltpu.sync_copy(w_hbm, wv)
    pltpu.sync_copy(b_hbm, bv)

    lanes = lax.iota(jnp.int32, 16)
    zeros = jnp.zeros((16,), jnp.int32)
    ones = jnp.ones((16,), jnp.int32)

    for j in range(_NCHUNK):
        for k in range(_CHUNK // 16):
            r = lanes + (j * _CHUNK + k * 16)
            sl = pl.ds(k * 16, 16)
            uidx_v[j, sl] = plsc.load_gather(xv, [r, zeros])
            itidx_v[j, sl] = plsc.load_gather(xv, [r, ones]) + _FIELD0

    copies = []
    for j in range(_NCHUNK):
        dst = urows_v.at[pl.ds(j * _CHUNK, _CHUNK)]
        copies.append(pltpu.async_copy(table_hbm.at[uidx_v.at[j]], dst, sem))
        dst = itrows_v.at[pl.ds(j * _CHUNK, _CHUNK)]
        copies.append(pltpu.async_copy(table_hbm.at[itidx_v.at[j]], dst, sem))
    for c in copies:
        c.wait()

    w0 = plsc.load_gather(wv, [lanes, zeros])
    w1 = plsc.load_gather(wv, [lanes + 16, zeros])
    bvec = jnp.full((16,), plsc.load_gather(bv, [zeros])[0], jnp.float32)

    def body(blk, _):
        i0 = blk * 16
        res = jnp.zeros((16,), jnp.float32)
        for r in range(16):
            i = i0 + r
            u0 = urows_v[i, pl.ds(0, 16)]
            u1 = urows_v[i, pl.ds(16, 16)]
            t0 = itrows_v[i, pl.ds(0, 16)]
            t1 = itrows_v[i, pl.ds(16, 16)]
            s = u0 * t0 * w0 + u1 * t1 * w1
            res = jnp.where(lanes == r, jnp.sum(s), res)
        out_v[pl.ds(i0, 16)] = jnp.maximum(res + bvec, 0.0)
        return 0

    lax.fori_loop(0, _BPW // 16, body, 0)

    pltpu.sync_copy(out_v, out_hbm.at[pl.ds(base, _BPW)])


@jax.jit
def kernel(x, table, W, b):
    mesh = plsc.VectorSubcoreMesh(core_axis_name="c", subcore_axis_name="s")
    run = pl.kernel(
        _gmf_body,
        mesh=mesh,
        compiler_params=pltpu.CompilerParams(
            needs_layout_passes=False, use_tc_tiling_on_sc=False),
        out_type=jax.ShapeDtypeStruct((_BATCH,), jnp.float32),
        scratch_types=[
            pltpu.VMEM((_BPW, 2), jnp.int32),
            pltpu.VMEM((_NCHUNK, _CHUNK), jnp.int32),
            pltpu.VMEM((_NCHUNK, _CHUNK), jnp.int32),
            pltpu.VMEM((_BPW, _EMBED), jnp.float32),
            pltpu.VMEM((_BPW, _EMBED), jnp.float32),
            pltpu.VMEM((_BPW,), jnp.float32),
            pltpu.VMEM((_EMBED, 1), jnp.float32),
            pltpu.VMEM((1,), jnp.float32),
            pltpu.SemaphoreType.DMA,
        ],
    )
    y = run(x, table, W, b)
    return y.reshape(_BATCH, 1)


# speedup vs baseline: 1.6428x; 1.6428x over previous
import jax
import jax.numpy as jnp
from jax import lax
from jax.experimental import pallas as pl
from jax.experimental.pallas import tpu as pltpu
from jax.experimental.pallas import tpu_sc as plsc

_FIELD0 = 1000000
_EMBED = 32
_BATCH = 16384
_NC = 2
_NS = 16
_NW = _NC * _NS
_BPW = _BATCH // _NW
_CHUNK = 256
_NCH = _BPW // _CHUNK


def _gmf_body(x_hbm, table_hbm, wb_hbm, out_hbm,
              xv, urows_v, itrows_v, out_v, wb_v, sem):
    wid = lax.axis_index("s") * _NC + lax.axis_index("c")
    base = wid * _BPW

    pltpu.sync_copy(x_hbm.at[pl.ds(2 * base, 2 * _BPW)], xv)
    pltpu.sync_copy(wb_hbm, wb_v)

    lanes = lax.iota(jnp.int32, 16)
    w0 = wb_v[pl.ds(0, 16)]
    w1 = wb_v[pl.ds(16, 16)]
    bvec = jnp.full((16,), wb_v[pl.ds(24, 16)][8], jnp.float32)

    for ch in range(_NCH):
        def enq(g, _):
            r0 = ch * _CHUNK + g * 16
            uvals = plsc.load_gather(xv, [(r0 + lanes) * 2])
            itvals = plsc.load_gather(xv, [(r0 + lanes) * 2 + 1]) + _FIELD0
            for k in range(16):
                dst = urows_v.at[pl.ds(g * 16 + k, 1)]
                pltpu.async_copy(table_hbm.at[pl.ds(uvals[k], 1)], dst, sem)
                dst = itrows_v.at[pl.ds(g * 16 + k, 1)]
                pltpu.async_copy(table_hbm.at[pl.ds(itvals[k], 1)], dst, sem)
            return 0

        lax.fori_loop(0, _CHUNK // 16, enq, 0)
        pltpu.make_async_copy(table_hbm.at[pl.ds(0, _CHUNK)], urows_v, sem).wait()
        pltpu.make_async_copy(table_hbm.at[pl.ds(0, _CHUNK)], itrows_v, sem).wait()

        def comp(blk, _):
            i0 = blk * 16
            res = jnp.zeros((16,), jnp.float32)
            for r in range(16):
                i = i0 + r
                u0 = urows_v[i, pl.ds(0, 16)]
                u1 = urows_v[i, pl.ds(16, 16)]
                t0 = itrows_v[i, pl.ds(0, 16)]
                t1 = itrows_v[i, pl.ds(16, 16)]
                s = u0 * t0 * w0 + u1 * t1 * w1
                res = jnp.where(lanes == r, jnp.sum(s), res)
            out_v[pl.ds(ch * _CHUNK + i0, 16)] = jnp.maximum(res + bvec, 0.0)
            return 0

        lax.fori_loop(0, _CHUNK // 16, comp, 0)

    pltpu.sync_copy(out_v, out_hbm.at[pl.ds(base, _BPW)])


@jax.jit
def kernel(x, table, W, b):
    x1 = x.reshape(-1)
    wb = jnp.concatenate([W.reshape(-1), b, jnp.zeros((7,), jnp.float32)])

    mesh = plsc.VectorSubcoreMesh(core_axis_name="c", subcore_axis_name="s")
    run = pl.kernel(
        _gmf_body,
        mesh=mesh,
        compiler_params=pltpu.CompilerParams(
            needs_layout_passes=False, use_tc_tiling_on_sc=True),
        out_type=jax.ShapeDtypeStruct((_BATCH,), jnp.float32),
        scratch_types=[
            pltpu.VMEM((2 * _BPW,), jnp.int32),
            pltpu.VMEM((_CHUNK, _EMBED), jnp.float32),
            pltpu.VMEM((_CHUNK, _EMBED), jnp.float32),
            pltpu.VMEM((_BPW,), jnp.float32),
            pltpu.VMEM((40,), jnp.float32),
            pltpu.SemaphoreType.DMA,
        ],
    )
    y = run(x1, table, wb)
    return y.reshape(_BATCH, 1)
